# Initial kernel scaffold; baseline (speedup 1.0000x reference)
import jax
import jax.numpy as jnp
from jax.experimental import pallas as pl


def kernel(output_features, y_truth, feature_centers):
    raise NotImplementedError("write your pallas kernel here")


# speedup vs baseline: 1.1916x; 1.1916x over previous
import jax
import jax.numpy as jnp
from jax import lax
from jax.experimental import pallas as pl
from jax.experimental.pallas import tpu as pltpu
from jax.experimental.pallas import tpu_sc as plsc

_FEAT = 128
_BATCH = 16384
_LAMDA = 1.0
_SCALE = 1.0
_NC = 2
_NS = 16
_NW = _NC * _NS
_RPW = _BATCH // _NW
_CHUNK = 128
_NCHUNK = _RPW // _CHUNK
_LANES = 16
_JG = _FEAT // _LANES


def _sc_body(x_hbm, y_hbm, table_hbm, out_hbm,
             idx_v, feat_v, rows_v, acc_v,
             sem_g0, sem_g1, sem_f0, sem_f1):
  cid = lax.axis_index("c")
  sid = lax.axis_index("s")
  wid = cid * _NS + sid

  sem_g = (sem_g0, sem_g1)
  sem_f = (sem_f0, sem_f1)

  pltpu.sync_copy(y_hbm.at[wid], idx_v)

  def start(c):
    b = c % 2
    g = pltpu.async_copy(table_hbm.at[idx_v.at[c]], rows_v.at[b], sem_g[b])
    f = pltpu.async_copy(x_hbm.at[wid, c], feat_v.at[b], sem_f[b])
    return g, f

  accs = tuple(jnp.zeros((_LANES,), jnp.float32) for _ in range(_JG))
  pending = start(0)
  for c in range(_NCHUNK):
    b = c % 2
    pending[0].wait()
    pending[1].wait()
    if c + 1 < _NCHUNK:
      pending = start(c + 1)

    @plsc.parallel_loop(0, _CHUNK, carry=accs, unroll=2)
    def _row(r, a):
      out = []
      for j in range(_JG):
        d = (feat_v[b, r, pl.ds(j * _LANES, _LANES)]
             - rows_v[b, r, pl.ds(j * _LANES, _LANES)])
        out.append(a[j] + d * d)
      return tuple(out)

    accs = _row

  total = accs[0]
  for j in range(1, _JG):
    total = total + accs[j]

  s = total[0]
  for i in range(1, _LANES):
    s = s + total[i]
  acc_v[...] = jnp.full((_LANES,), s, jnp.float32)
  pltpu.sync_copy(acc_v, out_hbm.at[wid])


def kernel(output_features, y_truth, feature_centers):
  x = output_features.reshape(_NW, _NCHUNK, _CHUNK, _FEAT)
  y = y_truth.astype(jnp.int32).reshape(_NW, _NCHUNK, _CHUNK)

  mesh = plsc.VectorSubcoreMesh(core_axis_name="c", subcore_axis_name="s")
  out = pl.kernel(
      _sc_body,
      out_type=jax.ShapeDtypeStruct((_NW, _LANES), jnp.float32),
      mesh=mesh,
      scratch_types=[
          pltpu.VMEM((_NCHUNK, _CHUNK), jnp.int32),
          pltpu.VMEM((2, _CHUNK, _FEAT), jnp.float32),
          pltpu.VMEM((2, _CHUNK, _FEAT), jnp.float32),
          pltpu.VMEM((_LANES,), jnp.float32),
          pltpu.SemaphoreType.DMA,
          pltpu.SemaphoreType.DMA,
          pltpu.SemaphoreType.DMA,
          pltpu.SemaphoreType.DMA,
      ],
  )(x, y, feature_centers)

  factor = _LAMDA * 0.5 * _SCALE / _BATCH
  return jnp.sum(out[:, 0]) * jnp.float32(factor)


# speedup vs baseline: 1.1930x; 1.0011x over previous
import jax
import jax.numpy as jnp
from jax import lax
from jax.experimental import pallas as pl
from jax.experimental.pallas import tpu as pltpu
from jax.experimental.pallas import tpu_sc as plsc

_FEAT = 128
_BATCH = 16384
_LAMDA = 1.0
_SCALE = 1.0
_NC = 2
_NS = 16
_NW = _NC * _NS
_RPW = _BATCH // _NW
_CHUNK = 128
_NCHUNK = _RPW // _CHUNK
_LANES = 16
_JG = _FEAT // _LANES


def _sc_body(x_hbm, y_hbm, table_hbm, out_hbm,
             idx_v, feat_v, rows_v, acc_v,
             sem_g0, sem_g1, sem_f0, sem_f1):
  cid = lax.axis_index("c")
  sid = lax.axis_index("s")
  wid = cid * _NS + sid

  sem_g = (sem_g0, sem_g1)
  sem_f = (sem_f0, sem_f1)

  pltpu.sync_copy(y_hbm.at[wid], idx_v)

  def start(c):
    b = c % 2
    g = pltpu.async_copy(table_hbm.at[idx_v.at[c]], rows_v.at[b], sem_g[b])
    f = pltpu.async_copy(x_hbm.at[wid, c], feat_v.at[b], sem_f[b])
    return g, f

  accs = tuple(jnp.zeros((_LANES,), jnp.float32) for _ in range(_JG))
  pending = start(0)
  for c in range(_NCHUNK):
    b = c % 2
    pending[0].wait()
    pending[1].wait()
    if c + 1 < _NCHUNK:
      pending = start(c + 1)

    @plsc.parallel_loop(0, _CHUNK, carry=accs, unroll=4)
    def _row(r, a):
      out = []
      for j in range(_JG):
        d = (feat_v[b, r, pl.ds(j * _LANES, _LANES)]
             - rows_v[b, r, pl.ds(j * _LANES, _LANES)])
        out.append(a[j] + d * d)
      return tuple(out)

    accs = _row

  total = accs[0]
  for j in range(1, _JG):
    total = total + accs[j]

  s = total[0]
  for i in range(1, _LANES):
    s = s + total[i]
  acc_v[...] = jnp.full((_LANES,), s, jnp.float32)
  pltpu.sync_copy(acc_v, out_hbm.at[wid])


def kernel(output_features, y_truth, feature_centers):
  x = output_features.reshape(_NW, _NCHUNK, _CHUNK, _FEAT)
  y = y_truth.astype(jnp.int32).reshape(_NW, _NCHUNK, _CHUNK)

  mesh = plsc.VectorSubcoreMesh(core_axis_name="c", subcore_axis_name="s")
  out = pl.kernel(
      _sc_body,
      out_type=jax.ShapeDtypeStruct((_NW, _LANES), jnp.float32),
      mesh=mesh,
      scratch_types=[
          pltpu.VMEM((_NCHUNK, _CHUNK), jnp.int32),
          pltpu.VMEM((2, _CHUNK, _FEAT), jnp.float32),
          pltpu.VMEM((2, _CHUNK, _FEAT), jnp.float32),
          pltpu.VMEM((_LANES,), jnp.float32),
          pltpu.SemaphoreType.DMA,
          pltpu.SemaphoreType.DMA,
          pltpu.SemaphoreType.DMA,
          pltpu.SemaphoreType.DMA,
      ],
  )(x, y, feature_centers)

  factor = _LAMDA * 0.5 * _SCALE / _BATCH
  return jnp.sum(out[:, 0]) * jnp.float32(factor)
